# Initial kernel scaffold; baseline (speedup 1.0000x reference)
import jax
import jax.numpy as jnp
from jax.experimental import pallas as pl


def kernel(x, x_points, y_points):
    raise NotImplementedError("write your pallas kernel here")


# speedup vs baseline: 2180.4069x; 2180.4069x over previous
import functools

import jax
import jax.numpy as jnp
from jax import lax
from jax.experimental import pallas as pl
from jax.experimental.pallas import tpu as pltpu
from jax.experimental.pallas import tpu_sc as plsc

N_KNOTS = 1024
M_BUCKETS = 8192
LANES = 16
CHUNK = 32768


def _build_tables(x_points, y_points):
    f32 = jnp.float32
    n = N_KNOTS
    h = x_points[1:] - x_points[:-1]
    dy = (y_points[1:] - y_points[:-1]) / h

    lo = jnp.concatenate([jnp.zeros((1,), f32), h / 6.0])
    up = jnp.concatenate([h / 6.0, jnp.zeros((1,), f32)])
    idx = jnp.arange(n)
    interior = (idx >= 1) & (idx <= n - 2)
    lo = jnp.where(interior, lo, 0.0)
    up = jnp.where(interior, up, 0.0)
    diag = jnp.where(interior, 2.0 * (lo + up), 1.0)
    rhs = jnp.zeros((n,), f32).at[1:-1].set(dy[1:] - dy[:-1])
    z = rhs / diag
    zero1 = jnp.zeros((1,), f32)
    for _ in range(40):
        zm = jnp.concatenate([zero1, z[:-1]])
        zp = jnp.concatenate([z[1:], zero1])
        z = (rhs - lo * zm - up * zp) / diag
    d2y = z

    c0 = d2y[:-1]
    c1 = d2y[1:]
    p1 = dy - h * (2.0 * c0 + c1) / 6.0
    p2 = c0 / 2.0
    p3 = (c1 - c0) / (6.0 * h)
    pad = jnp.zeros((1,), f32)
    p1 = jnp.concatenate([p1, pad])
    p2 = jnp.concatenate([p2, pad])
    p3 = jnp.concatenate([p3, pad])

    x0g = x_points[0]
    xng = x_points[-1]
    invw = f32(M_BUCKETS) / (xng - x0g)
    edges = x0g + jnp.arange(M_BUCKETS, dtype=f32) * ((xng - x0g) / f32(M_BUCKETS))
    cnt = jnp.sum(x_points[None, :] <= edges[:, None], axis=1).astype(jnp.int32)
    bucket_s = jnp.clip(cnt - 1, 0, n - 2)
    return p1, p2, p3, bucket_s, x0g, invw


def _spline_sc_body(x_hbm, bs_hbm, xp_hbm, yp_hbm, p1_hbm, p2_hbm, p3_hbm,
                    par_hbm, out_hbm,
                    bs_v, xp_v, yp_v, p1_v, p2_v, p3_v, par_v, xbuf, obuf,
                    *, per_tile, num_cores):
    wid = lax.axis_index("s") * num_cores + lax.axis_index("c")
    base = wid * per_tile

    pltpu.sync_copy(bs_hbm, bs_v)
    pltpu.sync_copy(xp_hbm, xp_v)
    pltpu.sync_copy(yp_hbm, yp_v)
    pltpu.sync_copy(p1_hbm, p1_v)
    pltpu.sync_copy(p2_hbm, p2_v)
    pltpu.sync_copy(p3_hbm, p3_v)
    pltpu.sync_copy(par_hbm, par_v)

    x0v = par_v[pl.ds(0, LANES)]
    invwv = par_v[pl.ds(LANES, LANES)]
    one = jnp.ones((LANES,), jnp.int32)
    zeroi = jnp.zeros((LANES,), jnp.int32)
    maxj = jnp.full((LANES,), M_BUCKETS - 1, jnp.int32)
    maxi = jnp.full((LANES,), N_KNOTS - 2, jnp.int32)

    def chunk_body(k, _):
        cbase = base + k * CHUNK
        pltpu.sync_copy(x_hbm.at[pl.ds(cbase, CHUNK)], xbuf)

        def vec_body(v, _):
            off = v * (4 * LANES)
            for u in range(4):
                xv = xbuf[pl.ds(off + u * LANES, LANES)]
                t0 = (xv - x0v) * invwv
                j = jnp.minimum(jnp.maximum(t0.astype(jnp.int32), zeroi), maxj)
                s = plsc.load_gather(bs_v, [j])
                xs1 = plsc.load_gather(xp_v, [s + one])
                i = s + jnp.where(xv >= xs1, one, zeroi)
                i = jnp.minimum(i, maxi)
                x0 = plsc.load_gather(xp_v, [i])
                y0 = plsc.load_gather(yp_v, [i])
                q1 = plsc.load_gather(p1_v, [i])
                q2 = plsc.load_gather(p2_v, [i])
                q3 = plsc.load_gather(p3_v, [i])
                t = xv - x0
                res = y0 + t * (q1 + t * (q2 + t * q3))
                obuf[pl.ds(off + u * LANES, LANES)] = res
            return 0

        lax.fori_loop(0, CHUNK // (4 * LANES), vec_body, 0)
        pltpu.sync_copy(obuf, out_hbm.at[pl.ds(cbase, CHUNK)])
        return 0

    lax.fori_loop(0, per_tile // CHUNK, chunk_body, 0)


def kernel(x, x_points, y_points):
    n_eval = x.shape[0]
    info = plsc.get_sparse_core_info()
    num_workers = info.num_cores * info.num_subcores
    assert n_eval % (num_workers * CHUNK) == 0, n_eval
    per_tile = n_eval // num_workers

    x_points = x_points.astype(jnp.float32)
    y_points = y_points.astype(jnp.float32)
    p1, p2, p3, bucket_s, x0g, invw = _build_tables(x_points, y_points)
    params = jnp.concatenate([
        jnp.full((LANES,), x0g, jnp.float32),
        jnp.full((LANES,), invw, jnp.float32),
    ])

    mesh = plsc.VectorSubcoreMesh(core_axis_name="c", subcore_axis_name="s")
    f32 = jnp.float32
    run = pl.kernel(
        functools.partial(_spline_sc_body, per_tile=per_tile,
                          num_cores=info.num_cores),
        out_type=jax.ShapeDtypeStruct((n_eval,), f32),
        mesh=mesh,
        compiler_params=pltpu.CompilerParams(needs_layout_passes=False),
        scratch_types=[
            pltpu.VMEM((M_BUCKETS,), jnp.int32),
            pltpu.VMEM((N_KNOTS,), f32),
            pltpu.VMEM((N_KNOTS,), f32),
            pltpu.VMEM((N_KNOTS,), f32),
            pltpu.VMEM((N_KNOTS,), f32),
            pltpu.VMEM((N_KNOTS,), f32),
            pltpu.VMEM((2 * LANES,), f32),
            pltpu.VMEM((CHUNK,), f32),
            pltpu.VMEM((CHUNK,), f32),
        ],
    )
    return run(x, bucket_s, x_points, y_points, p1, p2, p3, params)


# speedup vs baseline: 7825.9850x; 3.5892x over previous
import functools

import jax
import jax.numpy as jnp
from jax import lax
from jax.experimental import pallas as pl
from jax.experimental.pallas import tpu as pltpu
from jax.experimental.pallas import tpu_sc as plsc

N_KNOTS = 1024
M_BUCKETS = 8192
LANES = 16
CHUNK = 32768


def _build_tables(x_points, y_points):
    f32 = jnp.float32
    n = N_KNOTS
    h = x_points[1:] - x_points[:-1]
    dy = (y_points[1:] - y_points[:-1]) / h

    lo = jnp.concatenate([jnp.zeros((1,), f32), h / 6.0])
    up = jnp.concatenate([h / 6.0, jnp.zeros((1,), f32)])
    idx = jnp.arange(n)
    interior = (idx >= 1) & (idx <= n - 2)
    lo = jnp.where(interior, lo, 0.0)
    up = jnp.where(interior, up, 0.0)
    diag = jnp.where(interior, 2.0 * (lo + up), 1.0)
    rhs = jnp.zeros((n,), f32).at[1:-1].set(dy[1:] - dy[:-1])
    z = rhs / diag
    zero1 = jnp.zeros((1,), f32)
    for _ in range(40):
        zm = jnp.concatenate([zero1, z[:-1]])
        zp = jnp.concatenate([z[1:], zero1])
        z = (rhs - lo * zm - up * zp) / diag
    d2y = z

    c0 = d2y[:-1]
    c1 = d2y[1:]
    p1 = dy - h * (2.0 * c0 + c1) / 6.0
    p2 = c0 / 2.0
    p3 = (c1 - c0) / (6.0 * h)
    pad = jnp.zeros((1,), f32)
    p1 = jnp.concatenate([p1, pad])
    p2 = jnp.concatenate([p2, pad])
    p3 = jnp.concatenate([p3, pad])

    x0g = x_points[0]
    xng = x_points[-1]
    invw = f32(M_BUCKETS) / (xng - x0g)
    edges = x0g + jnp.arange(M_BUCKETS, dtype=f32) * ((xng - x0g) / f32(M_BUCKETS))
    cnt = jnp.sum(x_points[None, :] <= edges[:, None], axis=1).astype(jnp.int32)
    bucket_s = jnp.clip(cnt - 1, 0, n - 2)
    return p1, p2, p3, bucket_s, x0g, invw


def _spline_sc_body(x_hbm, bs_hbm, xp_hbm, yp_hbm, p1_hbm, p2_hbm, p3_hbm,
                    par_hbm, out_hbm,
                    bs_v, xp_v, yp_v, p1_v, p2_v, p3_v, par_v, xbuf, obuf,
                    *, per_tile, num_cores):
    wid = lax.axis_index("s") * num_cores + lax.axis_index("c")
    base = wid * per_tile

    pltpu.sync_copy(bs_hbm, bs_v)
    pltpu.sync_copy(xp_hbm, xp_v)
    pltpu.sync_copy(yp_hbm, yp_v)
    pltpu.sync_copy(p1_hbm, p1_v)
    pltpu.sync_copy(p2_hbm, p2_v)
    pltpu.sync_copy(p3_hbm, p3_v)
    pltpu.sync_copy(par_hbm, par_v)

    x0v = par_v[pl.ds(0, LANES)]
    invwv = par_v[pl.ds(LANES, LANES)]
    one = jnp.ones((LANES,), jnp.int32)
    zeroi = jnp.zeros((LANES,), jnp.int32)
    maxj = jnp.full((LANES,), M_BUCKETS - 1, jnp.int32)
    maxi = jnp.full((LANES,), N_KNOTS - 2, jnp.int32)

    def chunk_body(k, _):
        cbase = base + k * CHUNK
        pltpu.sync_copy(x_hbm.at[pl.ds(cbase, CHUNK)], xbuf)

        @plsc.parallel_loop(0, CHUNK, step=LANES, unroll=8)
        def _(off):
            xv = xbuf[pl.ds(off, LANES)]
            t0 = (xv - x0v) * invwv
            j = jnp.minimum(jnp.maximum(t0.astype(jnp.int32), zeroi), maxj)
            s = plsc.load_gather(bs_v, [j])
            xs1 = plsc.load_gather(xp_v, [s + one])
            i = s + jnp.where(xv >= xs1, one, zeroi)
            i = jnp.minimum(i, maxi)
            x0 = plsc.load_gather(xp_v, [i])
            y0 = plsc.load_gather(yp_v, [i])
            q1 = plsc.load_gather(p1_v, [i])
            q2 = plsc.load_gather(p2_v, [i])
            q3 = plsc.load_gather(p3_v, [i])
            t = xv - x0
            res = y0 + t * (q1 + t * (q2 + t * q3))
            obuf[pl.ds(off, LANES)] = res

        pltpu.sync_copy(obuf, out_hbm.at[pl.ds(cbase, CHUNK)])
        return 0

    lax.fori_loop(0, per_tile // CHUNK, chunk_body, 0)


def kernel(x, x_points, y_points):
    n_eval = x.shape[0]
    info = plsc.get_sparse_core_info()
    num_workers = info.num_cores * info.num_subcores
    assert n_eval % (num_workers * CHUNK) == 0, n_eval
    per_tile = n_eval // num_workers

    x_points = x_points.astype(jnp.float32)
    y_points = y_points.astype(jnp.float32)
    p1, p2, p3, bucket_s, x0g, invw = _build_tables(x_points, y_points)
    params = jnp.concatenate([
        jnp.full((LANES,), x0g, jnp.float32),
        jnp.full((LANES,), invw, jnp.float32),
    ])

    mesh = plsc.VectorSubcoreMesh(core_axis_name="c", subcore_axis_name="s")
    f32 = jnp.float32
    run = pl.kernel(
        functools.partial(_spline_sc_body, per_tile=per_tile,
                          num_cores=info.num_cores),
        out_type=jax.ShapeDtypeStruct((n_eval,), f32),
        mesh=mesh,
        compiler_params=pltpu.CompilerParams(needs_layout_passes=False),
        scratch_types=[
            pltpu.VMEM((M_BUCKETS,), jnp.int32),
            pltpu.VMEM((N_KNOTS,), f32),
            pltpu.VMEM((N_KNOTS,), f32),
            pltpu.VMEM((N_KNOTS,), f32),
            pltpu.VMEM((N_KNOTS,), f32),
            pltpu.VMEM((N_KNOTS,), f32),
            pltpu.VMEM((2 * LANES,), f32),
            pltpu.VMEM((CHUNK,), f32),
            pltpu.VMEM((CHUNK,), f32),
        ],
    )
    return run(x, bucket_s, x_points, y_points, p1, p2, p3, params)
